# baseline (speedup 1.0000x reference)
import jax
import jax.numpy as jnp
from jax import lax
from jax.experimental import pallas as pl
from jax.experimental.pallas import tpu as pltpu
from jax.experimental.pallas import tpu_sc as plsc

B = 16384
NN = 13
NCAT = 26
CARD = 100000
D = 32
NTOK = 1 + NN + NCAT

NC = 2
NS = 16
NW = NC * NS
BW = B // NW
CB = 64
NCHUNK = BW // CB


def _sc_body(numT, catT, w_hbm, bias_hbm, cls_hbm, tables, out,
             craw, nraw, cat_v, numcls_v, w_v, bias_v, cls_v, gsem, wsem):
    wid = lax.axis_index("s") * NC + lax.axis_index("c")
    base = pl.multiple_of(wid * BW, BW)

    pltpu.sync_copy(w_hbm, w_v)
    pltpu.sync_copy(bias_hbm, bias_v)
    pltpu.sync_copy(cls_hbm, cls_v)

    cls0 = cls_v[pl.ds(0, 16)]
    cls1 = cls_v[pl.ds(16, 16)]

    def fill_cls(b, carry):
        numcls_v[0, b, pl.ds(0, 16)] = cls0
        numcls_v[0, b, pl.ds(16, 16)] = cls1
        return carry
    lax.fori_loop(0, CB, fill_cls, 0)

    def chunk(t, carry):
        b0 = pl.multiple_of(base + t * CB, CB)
        pltpu.sync_copy(catT.at[:, pl.ds(b0, CB)], craw)
        pltpu.sync_copy(numT.at[:, pl.ds(b0, CB)], nraw)

        gh = []
        for i in range(NCAT):
            for k in range(CB // 16):
                craw[i, pl.ds(16 * k, 16)] = craw[i, pl.ds(16 * k, 16)] + (i * CARD)
            gh.append(pltpu.async_copy(tables.at[craw.at[i]], cat_v.at[i], gsem))

        def jloop(j, carry):
            w0 = w_v[j, pl.ds(0, 16)]
            w1 = w_v[j, pl.ds(16, 16)]
            a0 = bias_v[j, pl.ds(0, 16)]
            a1 = bias_v[j, pl.ds(16, 16)]

            def bloop(b, c2):
                v = plsc.load_gather(
                    nraw, [jnp.broadcast_to(j, (16,)), jnp.broadcast_to(b, (16,))])
                numcls_v[1 + j, b, pl.ds(0, 16)] = v * w0 + a0
                numcls_v[1 + j, b, pl.ds(16, 16)] = v * w1 + a1
                return c2
            lax.fori_loop(0, CB, bloop, 0)
            return carry
        lax.fori_loop(0, NN, jloop, 0)

        for h in gh:
            h.wait()

        wh = []
        for j in range(1 + NN):
            wh.append(pltpu.async_copy(
                numcls_v.at[j], out.at[pl.ds(b0, CB), pl.ds(j * D, D)], wsem))
        for i in range(NCAT):
            wh.append(pltpu.async_copy(
                cat_v.at[i], out.at[pl.ds(b0, CB), pl.ds((1 + NN + i) * D, D)], wsem))
        for h in wh:
            h.wait()
        return carry
    lax.fori_loop(0, NCHUNK, chunk, 0)


def kernel(numeric, categorical, num_weight, num_bias, cat_tables, cls_token):
    numT = numeric.T
    catT = categorical.T
    tables = cat_tables.reshape(NCAT * CARD, D)
    cls = cls_token.reshape(D)
    mesh = plsc.VectorSubcoreMesh(core_axis_name="c", subcore_axis_name="s")
    fn = pl.kernel(
        _sc_body,
        out_type=jax.ShapeDtypeStruct((B, NTOK * D), jnp.float32),
        mesh=mesh,
        scratch_types=[
            pltpu.VMEM((NCAT, CB), jnp.int32),
            pltpu.VMEM((NN, CB), jnp.float32),
            pltpu.VMEM((NCAT, CB, D), jnp.float32),
            pltpu.VMEM((1 + NN, CB, D), jnp.float32),
            pltpu.VMEM((NN, D), jnp.float32),
            pltpu.VMEM((NN, D), jnp.float32),
            pltpu.VMEM((D,), jnp.float32),
            pltpu.SemaphoreType.DMA,
            pltpu.SemaphoreType.DMA,
        ],
        compiler_params=pltpu.CompilerParams(use_tc_tiling_on_sc=False,
                                             needs_layout_passes=False),
    )
    out = fn(numT, catT, num_weight, num_bias, cls, tables)
    return out.reshape(B, NTOK, D)
